# baseline (speedup 1.0000x reference)
import functools

import jax
import jax.numpy as jnp
from jax import lax
from jax.experimental import pallas as pl

_LOG2E = 1.4426950408889634


def _proj_body(feat_ref, wt_ref, alt_ref, art_ref, bl_ref, br_ref,
               seqa_ref, f1_ref, f2_ref):
    s = jnp.dot(feat_ref[...], wt_ref[...], preferred_element_type=jnp.float32)
    br1, d = s.shape
    seqa_ref[:, :d] = s.astype(jnp.bfloat16)
    col = lax.broadcasted_iota(jnp.int32, (br1, d), 1)
    seqa_ref[:, d:] = jnp.where(col == 0, 1.0, 0.0).astype(jnp.bfloat16)
    f1_ref[...] = (jnp.dot(s, alt_ref[...], preferred_element_type=jnp.float32)
                   + bl_ref[...]) * _LOG2E
    f2_ref[...] = (jnp.dot(s, art_ref[...], preferred_element_type=jnp.float32)
                   + br_ref[...]) * _LOG2E


def _half_attn(adj, f1, f2t, seqa, bias):
    big = 1e9 * _LOG2E
    logits = f1 + f2t
    lrelu = jnp.maximum(logits, 0.2 * logits)
    x = lrelu - big * (1.0 - adj)
    m = jnp.max(x, axis=1, keepdims=True)
    e = jnp.exp2(x - m).astype(jnp.bfloat16)
    va = jax.lax.dot_general(e, seqa, (((1,), (0,)), ((), ())),
                             preferred_element_type=jnp.float32)
    d = bias.shape[1]
    out = va[:, :d] / va[:, d:d + 1] + bias
    return jnp.where(out > 0, out, jnp.exp(jnp.minimum(out, 0.0)) - 1.0)


def _attn_body(adj_ref, adjb_ref, f1_ref, f1b_ref, f2t_ref, seqa_ref, bias_ref,
               out_ref, outb_ref):
    out_ref[...] = _half_attn(adj_ref[...], f1_ref[...], f2t_ref[...],
                              seqa_ref[...], bias_ref[...])
    outb_ref[...] = _half_attn(adjb_ref[...], f1b_ref[...], f2t_ref[...],
                               seqa_ref[...], bias_ref[...])


@jax.jit
def kernel(feat, adj, W, a_l, b_l, a_r, b_r, bias):
    n, d_in = feat.shape
    d_out = W.shape[0]

    br1 = 2000
    seqa, f1, f2 = pl.pallas_call(
        _proj_body,
        grid=(n // br1,),
        in_specs=[
            pl.BlockSpec((br1, d_in), lambda r: (r, 0)),
            pl.BlockSpec((d_in, d_out), lambda r: (0, 0)),
            pl.BlockSpec((d_out, 1), lambda r: (0, 0)),
            pl.BlockSpec((d_out, 1), lambda r: (0, 0)),
            pl.BlockSpec((1, 1), lambda r: (0, 0)),
            pl.BlockSpec((1, 1), lambda r: (0, 0)),
        ],
        out_specs=[
            pl.BlockSpec((br1, 2 * d_out), lambda r: (r, 0)),
            pl.BlockSpec((br1, 1), lambda r: (r, 0)),
            pl.BlockSpec((br1, 1), lambda r: (r, 0)),
        ],
        out_shape=[
            jax.ShapeDtypeStruct((n, 2 * d_out), jnp.bfloat16),
            jax.ShapeDtypeStruct((n, 1), jnp.float32),
            jax.ShapeDtypeStruct((n, 1), jnp.float32),
        ],
    )(feat, W.T, a_l.T, a_r.T, b_l.reshape(1, 1), b_r.reshape(1, 1))

    f2t = f2.reshape(1, n)

    br = 200
    half = n // (2 * br)
    out, outb = pl.pallas_call(
        _attn_body,
        grid=(half,),
        in_specs=[
            pl.BlockSpec((br, n), lambda r: (r, 0)),
            pl.BlockSpec((br, n), lambda r: (r + half, 0)),
            pl.BlockSpec((br, 1), lambda r: (r, 0)),
            pl.BlockSpec((br, 1), lambda r: (r + half, 0)),
            pl.BlockSpec((1, n), lambda r: (0, 0)),
            pl.BlockSpec((n, 2 * d_out), lambda r: (0, 0)),
            pl.BlockSpec((1, d_out), lambda r: (0, 0)),
        ],
        out_specs=[
            pl.BlockSpec((br, d_out), lambda r: (r, 0)),
            pl.BlockSpec((br, d_out), lambda r: (r, 0)),
        ],
        out_shape=[
            jax.ShapeDtypeStruct((n // 2, d_out), jnp.float32),
            jax.ShapeDtypeStruct((n // 2, d_out), jnp.float32),
        ],
    )(adj, adj, f1, f1, f2t, seqa, bias.reshape(1, d_out))

    return jnp.concatenate([out, outb], axis=0)


# speedup vs baseline: 1.3338x; 1.3338x over previous
import functools

import jax
import jax.numpy as jnp
from jax import lax
from jax.experimental import pallas as pl

_LOG2E = 1.4426950408889634


def _proj_body(feat_ref, wt_ref, alt_ref, art_ref, bl_ref, br_ref,
               seqa_ref, f1_ref, f2_ref, f2m_ref):
    r = pl.program_id(0)
    s = jnp.dot(feat_ref[...], wt_ref[...], preferred_element_type=jnp.float32)
    br1, d = s.shape
    seqa_ref[:, :d] = s.astype(jnp.bfloat16)
    col = lax.broadcasted_iota(jnp.int32, (br1, d), 1)
    seqa_ref[:, d:] = jnp.where(col == 0, 1.0, 0.0).astype(jnp.bfloat16)
    f1_ref[...] = (jnp.dot(s, alt_ref[...], preferred_element_type=jnp.float32)
                   + bl_ref[...]) * _LOG2E
    f2 = (jnp.dot(s, art_ref[...], preferred_element_type=jnp.float32)
          + br_ref[...]) * _LOG2E
    f2_ref[...] = f2
    bm = jnp.max(f2, axis=0, keepdims=True)

    @pl.when(r == 0)
    def _():
        f2m_ref[...] = bm

    @pl.when(r > 0)
    def _():
        f2m_ref[...] = jnp.maximum(f2m_ref[...], bm)


def _attn_body(adj_ref, f1_ref, f2t_ref, f2m_ref, seqa_ref, bias_ref, out_ref):
    f1 = f1_ref[...]
    q = f1 + f2m_ref[...]
    mrow = jnp.maximum(q, 0.2 * q)
    g = f1 - mrow
    h = -0.8 * mrow
    lm = g + f2t_ref[...]
    lr = jnp.maximum(lm, 0.2 * lm + h)
    arg = jnp.where(adj_ref[...] != 0.0, lr, -400.0)
    e = jnp.exp2(arg).astype(jnp.bfloat16)
    va = jax.lax.dot_general(e, seqa_ref[...], (((1,), (0,)), ((), ())),
                             preferred_element_type=jnp.float32)
    d = out_ref.shape[1]
    out = va[:, :d] / va[:, d:d + 1] + bias_ref[...]
    out_ref[...] = jnp.where(out > 0, out, jnp.exp(jnp.minimum(out, 0.0)) - 1.0)


@jax.jit
def kernel(feat, adj, W, a_l, b_l, a_r, b_r, bias):
    n, d_in = feat.shape
    d_out = W.shape[0]

    br1 = 2000
    seqa, f1, f2, f2m = pl.pallas_call(
        _proj_body,
        grid=(n // br1,),
        in_specs=[
            pl.BlockSpec((br1, d_in), lambda r: (r, 0)),
            pl.BlockSpec((d_in, d_out), lambda r: (0, 0)),
            pl.BlockSpec((d_out, 1), lambda r: (0, 0)),
            pl.BlockSpec((d_out, 1), lambda r: (0, 0)),
            pl.BlockSpec((1, 1), lambda r: (0, 0)),
            pl.BlockSpec((1, 1), lambda r: (0, 0)),
        ],
        out_specs=[
            pl.BlockSpec((br1, 2 * d_out), lambda r: (r, 0)),
            pl.BlockSpec((br1, 1), lambda r: (r, 0)),
            pl.BlockSpec((br1, 1), lambda r: (r, 0)),
            pl.BlockSpec((1, 1), lambda r: (0, 0)),
        ],
        out_shape=[
            jax.ShapeDtypeStruct((n, 2 * d_out), jnp.bfloat16),
            jax.ShapeDtypeStruct((n, 1), jnp.float32),
            jax.ShapeDtypeStruct((n, 1), jnp.float32),
            jax.ShapeDtypeStruct((1, 1), jnp.float32),
        ],
    )(feat, W.T, a_l.T, a_r.T, b_l.reshape(1, 1), b_r.reshape(1, 1))

    f2t = f2.reshape(1, n)

    br = 200
    out = pl.pallas_call(
        _attn_body,
        grid=(n // br,),
        in_specs=[
            pl.BlockSpec((br, n), lambda r: (r, 0)),
            pl.BlockSpec((br, 1), lambda r: (r, 0)),
            pl.BlockSpec((1, n), lambda r: (0, 0)),
            pl.BlockSpec((1, 1), lambda r: (0, 0)),
            pl.BlockSpec((n, 2 * d_out), lambda r: (0, 0)),
            pl.BlockSpec((1, d_out), lambda r: (0, 0)),
        ],
        out_specs=pl.BlockSpec((br, d_out), lambda r: (r, 0)),
        out_shape=jax.ShapeDtypeStruct((n, d_out), jnp.float32),
    )(adj, f1, f2t, f2m, seqa, bias.reshape(1, d_out))

    return out


# speedup vs baseline: 1.4704x; 1.1024x over previous
import functools

import jax
import jax.numpy as jnp
from jax import lax
from jax.experimental import pallas as pl

_LOG2E = 1.4426950408889634


def _proj_body(feat_ref, wt_ref, alt_ref, art_ref, bl_ref, br_ref,
               seqa_ref, f1_ref, f2_ref, f2m_ref):
    r = pl.program_id(0)
    s = jnp.dot(feat_ref[...], wt_ref[...], preferred_element_type=jnp.float32)
    br1, d = s.shape
    seqa_ref[:, :d] = s.astype(jnp.bfloat16)
    col = lax.broadcasted_iota(jnp.int32, (br1, d), 1)
    seqa_ref[:, d:] = jnp.where(col == 0, 1.0, 0.0).astype(jnp.bfloat16)
    f1_ref[...] = (jnp.dot(s, alt_ref[...], preferred_element_type=jnp.float32)
                   + bl_ref[...]) * _LOG2E
    f2 = (jnp.dot(s, art_ref[...], preferred_element_type=jnp.float32)
          + br_ref[...]) * _LOG2E
    f2_ref[...] = f2
    bm = jnp.max(f2, axis=0, keepdims=True)

    @pl.when(r == 0)
    def _():
        f2m_ref[...] = bm

    @pl.when(r > 0)
    def _():
        f2m_ref[...] = jnp.maximum(f2m_ref[...], bm)


def _attn_body(adj_ref, f1_ref, f2t_ref, f2m_ref, seqa_ref, bias_ref, out_ref):
    f1 = f1_ref[...]
    q = f1 + f2m_ref[...]
    mrow = jnp.maximum(q, 0.2 * q)
    g = f1 - mrow
    h = -0.8 * mrow
    lm = g + f2t_ref[...]
    lr = jnp.maximum(lm, 0.2 * lm + h)
    arg = jnp.where(adj_ref[...] != 0.0, lr, -400.0)
    e = jnp.exp2(arg).astype(jnp.bfloat16)
    va = jax.lax.dot_general(e, seqa_ref[...], (((1,), (0,)), ((), ())),
                             preferred_element_type=jnp.float32)
    d = out_ref.shape[1]
    out = va[:, :d] / va[:, d:d + 1] + bias_ref[...]
    out_ref[...] = jnp.where(out > 0, out, jnp.exp(jnp.minimum(out, 0.0)) - 1.0)


@jax.jit
def kernel(feat, adj, W, a_l, b_l, a_r, b_r, bias):
    n, d_in = feat.shape
    d_out = W.shape[0]

    br1 = 2000
    seqa, f1, f2, f2m = pl.pallas_call(
        _proj_body,
        grid=(n // br1,),
        in_specs=[
            pl.BlockSpec((br1, d_in), lambda r: (r, 0)),
            pl.BlockSpec((d_in, d_out), lambda r: (0, 0)),
            pl.BlockSpec((d_out, 1), lambda r: (0, 0)),
            pl.BlockSpec((d_out, 1), lambda r: (0, 0)),
            pl.BlockSpec((1, 1), lambda r: (0, 0)),
            pl.BlockSpec((1, 1), lambda r: (0, 0)),
        ],
        out_specs=[
            pl.BlockSpec((br1, 2 * d_out), lambda r: (r, 0)),
            pl.BlockSpec((br1, 1), lambda r: (r, 0)),
            pl.BlockSpec((br1, 1), lambda r: (r, 0)),
            pl.BlockSpec((1, 1), lambda r: (0, 0)),
        ],
        out_shape=[
            jax.ShapeDtypeStruct((n, 2 * d_out), jnp.bfloat16),
            jax.ShapeDtypeStruct((n, 1), jnp.float32),
            jax.ShapeDtypeStruct((n, 1), jnp.float32),
            jax.ShapeDtypeStruct((1, 1), jnp.float32),
        ],
    )(feat, W.T, a_l.T, a_r.T, b_l.reshape(1, 1), b_r.reshape(1, 1))

    f2t = f2.reshape(1, n)

    br = 400
    out = pl.pallas_call(
        _attn_body,
        grid=(n // br,),
        in_specs=[
            pl.BlockSpec((br, n), lambda r: (r, 0)),
            pl.BlockSpec((br, 1), lambda r: (r, 0)),
            pl.BlockSpec((1, n), lambda r: (0, 0)),
            pl.BlockSpec((1, 1), lambda r: (0, 0)),
            pl.BlockSpec((n, 2 * d_out), lambda r: (0, 0)),
            pl.BlockSpec((1, d_out), lambda r: (0, 0)),
        ],
        out_specs=pl.BlockSpec((br, d_out), lambda r: (r, 0)),
        out_shape=jax.ShapeDtypeStruct((n, d_out), jnp.float32),
    )(adj, f1, f2t, f2m, seqa, bias.reshape(1, d_out))

    return out


# speedup vs baseline: 1.5140x; 1.0297x over previous
import functools

import jax
import jax.numpy as jnp
from jax import lax
from jax.experimental import pallas as pl

_LOG2E = 1.4426950408889634


def _proj_body(feat_ref, wt_ref, alt_ref, art_ref, bl_ref, br_ref,
               seqa_ref, f1_ref, f2_ref, f2m_ref):
    r = pl.program_id(0)
    s = jnp.dot(feat_ref[...], wt_ref[...], preferred_element_type=jnp.float32)
    br1, d = s.shape
    seqa_ref[:, :d] = s.astype(jnp.bfloat16)
    col = lax.broadcasted_iota(jnp.int32, (br1, d), 1)
    seqa_ref[:, d:] = jnp.where(col == 0, 1.0, 0.0).astype(jnp.bfloat16)
    f1_ref[...] = (jnp.dot(s, alt_ref[...], preferred_element_type=jnp.float32)
                   + bl_ref[...]) * _LOG2E
    f2 = (jnp.dot(s, art_ref[...], preferred_element_type=jnp.float32)
          + br_ref[...]) * _LOG2E
    f2_ref[...] = f2
    bm = jnp.max(f2, axis=0, keepdims=True)

    @pl.when(r == 0)
    def _():
        f2m_ref[...] = bm

    @pl.when(r > 0)
    def _():
        f2m_ref[...] = jnp.maximum(f2m_ref[...], bm)


def _attn_body(adj_ref, f1_ref, f2t_ref, f2s_ref, f2m_ref, seqa_ref, bias_ref,
               out_ref):
    f1 = f1_ref[...]
    q = f1 + f2m_ref[...]
    mrow = jnp.maximum(q, 0.2 * q)
    g = f1 - mrow
    g2 = 0.2 * f1 - mrow
    lm = g + f2t_ref[...]
    lr = jnp.maximum(lm, g2 + f2s_ref[...])
    arg = jnp.where(adj_ref[...] != 0.0, lr, -400.0)
    e = jnp.exp2(arg).astype(jnp.bfloat16)
    va = jax.lax.dot_general(e, seqa_ref[...], (((1,), (0,)), ((), ())),
                             preferred_element_type=jnp.float32)
    d = out_ref.shape[1]
    out = va[:, :d] / va[:, d:d + 1] + bias_ref[...]
    out_ref[...] = jnp.where(out > 0, out, jnp.exp(jnp.minimum(out, 0.0)) - 1.0)


@jax.jit
def kernel(feat, adj, W, a_l, b_l, a_r, b_r, bias):
    n, d_in = feat.shape
    d_out = W.shape[0]

    br1 = 2000
    seqa, f1, f2, f2m = pl.pallas_call(
        _proj_body,
        grid=(n // br1,),
        in_specs=[
            pl.BlockSpec((br1, d_in), lambda r: (r, 0)),
            pl.BlockSpec((d_in, d_out), lambda r: (0, 0)),
            pl.BlockSpec((d_out, 1), lambda r: (0, 0)),
            pl.BlockSpec((d_out, 1), lambda r: (0, 0)),
            pl.BlockSpec((1, 1), lambda r: (0, 0)),
            pl.BlockSpec((1, 1), lambda r: (0, 0)),
        ],
        out_specs=[
            pl.BlockSpec((br1, 2 * d_out), lambda r: (r, 0)),
            pl.BlockSpec((br1, 1), lambda r: (r, 0)),
            pl.BlockSpec((br1, 1), lambda r: (r, 0)),
            pl.BlockSpec((1, 1), lambda r: (0, 0)),
        ],
        out_shape=[
            jax.ShapeDtypeStruct((n, 2 * d_out), jnp.bfloat16),
            jax.ShapeDtypeStruct((n, 1), jnp.float32),
            jax.ShapeDtypeStruct((n, 1), jnp.float32),
            jax.ShapeDtypeStruct((1, 1), jnp.float32),
        ],
    )(feat, W.T, a_l.T, a_r.T, b_l.reshape(1, 1), b_r.reshape(1, 1))

    f2t = f2.reshape(1, n)
    f2s = 0.2 * f2t

    br = 400
    out = pl.pallas_call(
        _attn_body,
        grid=(n // br,),
        in_specs=[
            pl.BlockSpec((br, n), lambda r: (r, 0)),
            pl.BlockSpec((br, 1), lambda r: (r, 0)),
            pl.BlockSpec((1, n), lambda r: (0, 0)),
            pl.BlockSpec((1, n), lambda r: (0, 0)),
            pl.BlockSpec((1, 1), lambda r: (0, 0)),
            pl.BlockSpec((n, 2 * d_out), lambda r: (0, 0)),
            pl.BlockSpec((1, d_out), lambda r: (0, 0)),
        ],
        out_specs=pl.BlockSpec((br, d_out), lambda r: (r, 0)),
        out_shape=jax.ShapeDtypeStruct((n, d_out), jnp.float32),
    )(adj, f1, f2t, f2s, f2m, seqa, bias.reshape(1, d_out))

    return out
